# baseline (speedup 1.0000x reference)
import jax
import jax.numpy as jnp
from jax import lax
from jax.experimental import pallas as pl
from jax.experimental.pallas import tpu as pltpu
from jax.experimental.pallas import tpu_sc as plsc

B = 16384
D = 64
H1 = 256
H2 = 128
NC = 2
NS = 16
NW = NC * NS
BPW = B // NW
CH = 128
NCH = BPW // CH

MIN_RATING = 0.5
MAX_RATING = 5.0


def _sc_gather_body(uid_hbm, mid_hbm, ut_hbm, mt_hbm, xo_hbm,
                    uidx_v, midx_v, urows_v, mrows_v, usem, msem):
    wid = lax.axis_index("s") * NC + lax.axis_index("c")
    base = wid * BPW
    pltpu.sync_copy(uid_hbm.at[wid], uidx_v)
    pltpu.sync_copy(mid_hbm.at[wid], midx_v)
    copies = []
    for c in range(NCH):
        copies.append(pltpu.async_copy(
            ut_hbm.at[uidx_v.at[c]], urows_v.at[pl.ds(c * CH, CH)], usem))
        copies.append(pltpu.async_copy(
            mt_hbm.at[midx_v.at[c]], mrows_v.at[pl.ds(c * CH, CH)], msem))
    for cp in copies:
        cp.wait()
    pltpu.sync_copy(urows_v, xo_hbm.at[pl.ds(base, BPW), pl.ds(0, D)])
    pltpu.sync_copy(mrows_v, xo_hbm.at[pl.ds(base, BPW), pl.ds(D, D)])


def _sc_gather(uid, mid, user_table, movie_table):
    mesh = plsc.VectorSubcoreMesh(
        core_axis_name="c", subcore_axis_name="s",
        num_cores=NC, num_subcores=NS)
    f = pl.kernel(
        _sc_gather_body,
        out_type=jax.ShapeDtypeStruct((B, 2 * D), jnp.bfloat16),
        mesh=mesh,
        scratch_types=[
            pltpu.VMEM((NCH, CH), jnp.int32),
            pltpu.VMEM((NCH, CH), jnp.int32),
            pltpu.VMEM((BPW, D), jnp.bfloat16),
            pltpu.VMEM((BPW, D), jnp.bfloat16),
            pltpu.SemaphoreType.DMA,
            pltpu.SemaphoreType.DMA,
        ],
        compiler_params=pltpu.CompilerParams(use_tc_tiling_on_sc=False),
    )
    return f(uid, mid, user_table, movie_table)


BB = 2048


def _mlp_body(w1_ref, b1_ref, w2_ref, b2_ref, w3_ref, b3_ref, x_ref, out_ref):
    h = jnp.dot(x_ref[...], w1_ref[...], preferred_element_type=jnp.float32)
    h = jnp.maximum(h + b1_ref[...], 0.0)
    h = jnp.dot(h.astype(jnp.bfloat16), w2_ref[...],
                preferred_element_type=jnp.float32)
    h = jnp.maximum(h + b2_ref[...], 0.0)
    o = jnp.sum(h * w3_ref[...], axis=1, keepdims=True) + b3_ref[...]
    out_ref[...] = MIN_RATING + (MAX_RATING - MIN_RATING) * jax.nn.sigmoid(o)


def _mlp(x, w1, b1, w2, b2, w3, b3):
    grid = B // BB
    wspec = lambda shape: pl.BlockSpec(shape, lambda i: (0, 0))
    return pl.pallas_call(
        _mlp_body,
        grid=(grid,),
        in_specs=[
            wspec((2 * D, H1)), wspec((1, H1)),
            wspec((H1, H2)), wspec((1, H2)), wspec((1, H2)), wspec((1, 1)),
            pl.BlockSpec((BB, 2 * D), lambda i: (i, 0)),
        ],
        out_specs=pl.BlockSpec((BB, 1), lambda i: (i, 0)),
        out_shape=jax.ShapeDtypeStruct((B, 1), jnp.float32),
    )(w1, b1, w2, b2, w3, b3, x)


def kernel(user_ids, movie_ids, user_table, movie_table, W1, b1, W2, b2, W3, b3):
    uid = user_ids.astype(jnp.int32).reshape(NW, NCH, CH)
    mid = movie_ids.astype(jnp.int32).reshape(NW, NCH, CH)
    ut16 = user_table.astype(jnp.bfloat16)
    mt16 = movie_table.astype(jnp.bfloat16)
    x = _sc_gather(uid, mid, ut16, mt16)
    out = _mlp(x, W1.T.astype(jnp.bfloat16), b1.reshape(1, H1),
               W2.T.astype(jnp.bfloat16), b2.reshape(1, H2), W3, b3.reshape(1, 1))
    return out.reshape(B)


# speedup vs baseline: 1.3004x; 1.3004x over previous
import jax
import jax.numpy as jnp
from jax import lax
from jax.experimental import pallas as pl
from jax.experimental.pallas import tpu as pltpu
from jax.experimental.pallas import tpu_sc as plsc

B = 16384
D = 64
H1 = 256
H2 = 128
NC = 2
NS = 16
NW = NC * NS
BPW = B // NW
CH = 128
NCH = BPW // CH

MIN_RATING = 0.5
MAX_RATING = 5.0


def _sc_gather_body(uid_hbm, mid_hbm, ut_hbm, mt_hbm, uo_hbm, mo_hbm,
                    uidx_v, midx_v, rows_v, usem):
    wid = lax.axis_index("s") * NC + lax.axis_index("c")
    base = wid * BPW
    pltpu.sync_copy(uid_hbm.at[wid], uidx_v)
    pltpu.sync_copy(mid_hbm.at[wid], midx_v)
    copies = []
    for c in range(NCH):
        copies.append(pltpu.async_copy(
            ut_hbm.at[uidx_v.at[c]], rows_v.at[pl.ds(c * CH, CH)], usem))
    for cp in copies:
        cp.wait()
    pltpu.sync_copy(rows_v, uo_hbm.at[pl.ds(base, BPW)])
    copies = []
    for c in range(NCH):
        copies.append(pltpu.async_copy(
            mt_hbm.at[midx_v.at[c]], rows_v.at[pl.ds(c * CH, CH)], usem))
    for cp in copies:
        cp.wait()
    pltpu.sync_copy(rows_v, mo_hbm.at[pl.ds(base, BPW)])


def _sc_gather(uhalf, mhalf, ut_pairs, mt_pairs):
    mesh = plsc.VectorSubcoreMesh(
        core_axis_name="c", subcore_axis_name="s",
        num_cores=NC, num_subcores=NS)
    f = pl.kernel(
        _sc_gather_body,
        out_type=(jax.ShapeDtypeStruct((B, 2 * D), jnp.float32),
                  jax.ShapeDtypeStruct((B, 2 * D), jnp.float32)),
        mesh=mesh,
        scratch_types=[
            pltpu.VMEM((NCH, CH), jnp.int32),
            pltpu.VMEM((NCH, CH), jnp.int32),
            pltpu.VMEM((BPW, 2 * D), jnp.float32),
            pltpu.SemaphoreType.DMA,
        ],
    )
    return f(uhalf, mhalf, ut_pairs, mt_pairs)


BB = 2048


def _mlp_body(w1u_ref, w1m_ref, b1_ref, w2_ref, b2_ref, w3_ref, b3_ref,
              up_ref, mp_ref, pu_ref, pm_ref, out_ref):
    lanes = lax.broadcasted_iota(jnp.int32, (1, 2 * D), 1)
    lo = (lanes < D).astype(jnp.float32)
    umask = lo * (1.0 - pu_ref[...]) + (1.0 - lo) * pu_ref[...]
    mmask = lo * (1.0 - pm_ref[...]) + (1.0 - lo) * pm_ref[...]
    h = jnp.dot(up_ref[...] * umask, w1u_ref[...],
                preferred_element_type=jnp.float32)
    h = h + jnp.dot(mp_ref[...] * mmask, w1m_ref[...],
                    preferred_element_type=jnp.float32)
    h = jnp.maximum(h + b1_ref[...], 0.0)
    h = jnp.dot(h, w2_ref[...], preferred_element_type=jnp.float32)
    h = jnp.maximum(h + b2_ref[...], 0.0)
    o = jnp.sum(h * w3_ref[...], axis=1, keepdims=True) + b3_ref[...]
    out_ref[...] = MIN_RATING + (MAX_RATING - MIN_RATING) * jax.nn.sigmoid(o)


def _mlp(up, mp, pu, pm, w1u, w1m, b1, w2, b2, w3, b3):
    grid = B // BB
    wspec = lambda shape: pl.BlockSpec(shape, lambda i: (0, 0))
    return pl.pallas_call(
        _mlp_body,
        grid=(grid,),
        in_specs=[
            wspec((2 * D, H1)), wspec((2 * D, H1)), wspec((1, H1)),
            wspec((H1, H2)), wspec((1, H2)), wspec((1, H2)), wspec((1, 1)),
            pl.BlockSpec((BB, 2 * D), lambda i: (i, 0)),
            pl.BlockSpec((BB, 2 * D), lambda i: (i, 0)),
            pl.BlockSpec((BB, 1), lambda i: (i, 0)),
            pl.BlockSpec((BB, 1), lambda i: (i, 0)),
        ],
        out_specs=pl.BlockSpec((BB, 1), lambda i: (i, 0)),
        out_shape=jax.ShapeDtypeStruct((B, 1), jnp.float32),
    )(w1u, w1m, b1, w2, b2, w3, b3, up, mp, pu, pm)


def kernel(user_ids, movie_ids, user_table, movie_table, W1, b1, W2, b2, W3, b3):
    uid = user_ids.astype(jnp.int32)
    mid = movie_ids.astype(jnp.int32)
    uhalf = (uid >> 1).reshape(NW, NCH, CH)
    mhalf = (mid >> 1).reshape(NW, NCH, CH)
    pu = (uid & 1).astype(jnp.float32).reshape(B, 1)
    pm = (mid & 1).astype(jnp.float32).reshape(B, 1)
    ut_pairs = user_table.reshape(user_table.shape[0] // 2, 2 * D)
    mt_pairs = movie_table.reshape(movie_table.shape[0] // 2, 2 * D)
    up, mp = _sc_gather(uhalf, mhalf, ut_pairs, mt_pairs)
    w1u = W1[:, :D].T
    w1m = W1[:, D:].T
    w1u2 = jnp.concatenate([w1u, w1u], axis=0)
    w1m2 = jnp.concatenate([w1m, w1m], axis=0)
    out = _mlp(up, mp, pu, pm, w1u2, w1m2, b1.reshape(1, H1),
               W2.T, b2.reshape(1, H2), W3, b3.reshape(1, 1))
    return out.reshape(B)


# speedup vs baseline: 1.6316x; 1.2547x over previous
import jax
import jax.numpy as jnp
from jax import lax
from jax.experimental import pallas as pl
from jax.experimental.pallas import tpu as pltpu
from jax.experimental.pallas import tpu_sc as plsc

B = 16384
D = 64
H1 = 256
H2 = 128
NC = 2
NS = 16
NW = NC * NS
BPW = B // NW
CH = 128
NCH = BPW // CH

MIN_RATING = 0.5
MAX_RATING = 5.0


def _sc_gather_body(uid_hbm, mid_hbm, ut_hbm, mt_hbm, uo_hbm, mo_hbm,
                    uidx_v, midx_v, rows_v, usem):
    wid = lax.axis_index("s") * NC + lax.axis_index("c")
    base = wid * BPW
    pltpu.sync_copy(uid_hbm.at[wid], uidx_v)
    pltpu.sync_copy(mid_hbm.at[wid], midx_v)
    copies = []
    for c in range(NCH):
        copies.append(pltpu.async_copy(
            ut_hbm.at[uidx_v.at[c]], rows_v.at[pl.ds(c * CH, CH)], usem))
    for cp in copies:
        cp.wait()
    pltpu.sync_copy(rows_v, uo_hbm.at[pl.ds(base, BPW)])
    copies = []
    for c in range(NCH):
        copies.append(pltpu.async_copy(
            mt_hbm.at[midx_v.at[c]], rows_v.at[pl.ds(c * CH, CH)], usem))
    for cp in copies:
        cp.wait()
    pltpu.sync_copy(rows_v, mo_hbm.at[pl.ds(base, BPW)])


def _sc_gather(uhalf, mhalf, ut_pairs, mt_pairs):
    mesh = plsc.VectorSubcoreMesh(
        core_axis_name="c", subcore_axis_name="s",
        num_cores=NC, num_subcores=NS)
    f = pl.kernel(
        _sc_gather_body,
        out_type=(jax.ShapeDtypeStruct((B, 2 * D), jnp.float32),
                  jax.ShapeDtypeStruct((B, 2 * D), jnp.float32)),
        mesh=mesh,
        scratch_types=[
            pltpu.VMEM((NCH, CH), jnp.int32),
            pltpu.VMEM((NCH, CH), jnp.int32),
            pltpu.VMEM((BPW, 2 * D), jnp.float32),
            pltpu.SemaphoreType.DMA,
        ],
    )
    return f(uhalf, mhalf, ut_pairs, mt_pairs)


RC = 2048


def _repack_body(in_ref, out_ref):
    t = in_ref[...].T
    t4 = t.reshape(RC // 16, 2, 8, D)
    e = t4[:, 0].reshape(RC // 2, D)
    o = t4[:, 1].reshape(RC // 2, D)
    out_ref[...] = jnp.concatenate([e, o], axis=1)


def _repack(tab_t, n_rows):
    grid = (n_rows + RC - 1) // RC
    return pl.pallas_call(
        _repack_body,
        grid=(grid,),
        in_specs=[pl.BlockSpec((D, RC), lambda i: (0, i))],
        out_specs=pl.BlockSpec((RC // 2, 2 * D), lambda i: (i, 0)),
        out_shape=jax.ShapeDtypeStruct((n_rows // 2, 2 * D), jnp.float32),
    )(tab_t)


BB = 2048


def _mlp_body(w1u_ref, w1m_ref, b1_ref, w2_ref, b2_ref, w3_ref, b3_ref,
              up_ref, mp_ref, pu_ref, pm_ref, out_ref):
    lanes = lax.broadcasted_iota(jnp.int32, (1, 2 * D), 1)
    lo = (lanes < D).astype(jnp.float32)
    umask = lo * (1.0 - pu_ref[...]) + (1.0 - lo) * pu_ref[...]
    mmask = lo * (1.0 - pm_ref[...]) + (1.0 - lo) * pm_ref[...]
    h = jnp.dot(up_ref[...] * umask, w1u_ref[...],
                preferred_element_type=jnp.float32)
    h = h + jnp.dot(mp_ref[...] * mmask, w1m_ref[...],
                    preferred_element_type=jnp.float32)
    h = jnp.maximum(h + b1_ref[...], 0.0)
    h = jnp.dot(h, w2_ref[...], preferred_element_type=jnp.float32)
    h = jnp.maximum(h + b2_ref[...], 0.0)
    o = jnp.sum(h * w3_ref[...], axis=1, keepdims=True) + b3_ref[...]
    out_ref[...] = MIN_RATING + (MAX_RATING - MIN_RATING) * jax.nn.sigmoid(o)


def _mlp(up, mp, pu, pm, w1u, w1m, b1, w2, b2, w3, b3):
    grid = B // BB
    wspec = lambda shape: pl.BlockSpec(shape, lambda i: (0, 0))
    return pl.pallas_call(
        _mlp_body,
        grid=(grid,),
        in_specs=[
            wspec((2 * D, H1)), wspec((2 * D, H1)), wspec((1, H1)),
            wspec((H1, H2)), wspec((1, H2)), wspec((1, H2)), wspec((1, 1)),
            pl.BlockSpec((BB, 2 * D), lambda i: (i, 0)),
            pl.BlockSpec((BB, 2 * D), lambda i: (i, 0)),
            pl.BlockSpec((BB, 1), lambda i: (i, 0)),
            pl.BlockSpec((BB, 1), lambda i: (i, 0)),
        ],
        out_specs=pl.BlockSpec((BB, 1), lambda i: (i, 0)),
        out_shape=jax.ShapeDtypeStruct((B, 1), jnp.float32),
    )(w1u, w1m, b1, w2, b2, w3, b3, up, mp, pu, pm)


def kernel(user_ids, movie_ids, user_table, movie_table, W1, b1, W2, b2, W3, b3):
    uid = user_ids.astype(jnp.int32)
    mid = movie_ids.astype(jnp.int32)
    uhalf = (((uid >> 4) << 3) | (uid & 7)).reshape(NW, NCH, CH)
    mhalf = (((mid >> 4) << 3) | (mid & 7)).reshape(NW, NCH, CH)
    pu = ((uid >> 3) & 1).astype(jnp.float32).reshape(B, 1)
    pm = ((mid >> 3) & 1).astype(jnp.float32).reshape(B, 1)
    ut_pairs = _repack(user_table.T, user_table.shape[0])
    mt_pairs = _repack(movie_table.T, movie_table.shape[0])
    up, mp = _sc_gather(uhalf, mhalf, ut_pairs, mt_pairs)
    w1u = W1[:, :D].T
    w1m = W1[:, D:].T
    w1u2 = jnp.concatenate([w1u, w1u], axis=0)
    w1m2 = jnp.concatenate([w1m, w1m], axis=0)
    out = _mlp(up, mp, pu, pm, w1u2, w1m2, b1.reshape(1, H1),
               W2.T, b2.reshape(1, H2), W3, b3.reshape(1, 1))
    return out.reshape(B)


# speedup vs baseline: 2.5447x; 1.5597x over previous
import jax
import jax.numpy as jnp
from jax import lax
from jax.experimental import pallas as pl
from jax.experimental.pallas import tpu as pltpu
from jax.experimental.pallas import tpu_sc as plsc

B = 16384
D = 64
H1 = 256
H2 = 128
NC = 2
NS = 16
NW = NC * NS
BPW = B // NW
CH = 128
NCH = BPW // CH

MIN_RATING = 0.5
MAX_RATING = 5.0

RC = 4096
RQ = RC // 4
HI = -65536


def _repack_body(in_ref, out_ref):
    blk = in_ref[...].astype(jnp.bfloat16)
    r = lax.broadcasted_iota(jnp.int32, (D, D), 0)
    c = lax.broadcasted_iota(jnp.int32, (D, D), 1)
    eye = (r == c).astype(jnp.bfloat16)
    t = lax.dot_general(blk, eye, (((0,), (0,)), ((), ())),
                        preferred_element_type=jnp.float32)
    ti = lax.bitcast_convert_type(t, jnp.int32)
    q0 = ti[0 * RQ:1 * RQ]
    q1 = ti[1 * RQ:2 * RQ]
    q2 = ti[2 * RQ:3 * RQ]
    q3 = ti[3 * RQ:4 * RQ]
    p01 = (q1 & HI) | lax.shift_right_logical(q0, 16)
    p23 = (q3 & HI) | lax.shift_right_logical(q2, 16)
    out_ref[...] = jnp.concatenate([p01, p23], axis=1)


def _repack(tab_t, n_rows):
    grid = (n_rows + RC - 1) // RC
    return pl.pallas_call(
        _repack_body,
        grid=(grid,),
        in_specs=[pl.BlockSpec((D, RC), lambda i: (0, i))],
        out_specs=pl.BlockSpec((RQ, 2 * D), lambda i: (i, 0)),
        out_shape=jax.ShapeDtypeStruct((n_rows // 4, 2 * D), jnp.int32),
    )(tab_t)


def _sc_gather_body(uid_hbm, mid_hbm, ut_hbm, mt_hbm, uo_hbm, mo_hbm,
                    uidx_v, midx_v, rows_v, usem):
    wid = lax.axis_index("s") * NC + lax.axis_index("c")
    base = wid * BPW
    pltpu.sync_copy(uid_hbm.at[wid], uidx_v)
    pltpu.sync_copy(mid_hbm.at[wid], midx_v)
    copies = []
    for c in range(NCH):
        copies.append(pltpu.async_copy(
            ut_hbm.at[uidx_v.at[c]], rows_v.at[pl.ds(c * CH, CH)], usem))
    for cp in copies:
        cp.wait()
    pltpu.sync_copy(rows_v, uo_hbm.at[pl.ds(base, BPW)])
    copies = []
    for c in range(NCH):
        copies.append(pltpu.async_copy(
            mt_hbm.at[midx_v.at[c]], rows_v.at[pl.ds(c * CH, CH)], usem))
    for cp in copies:
        cp.wait()
    pltpu.sync_copy(rows_v, mo_hbm.at[pl.ds(base, BPW)])


def _sc_gather(uq, mq, ut_pack, mt_pack):
    mesh = plsc.VectorSubcoreMesh(
        core_axis_name="c", subcore_axis_name="s",
        num_cores=NC, num_subcores=NS)
    f = pl.kernel(
        _sc_gather_body,
        out_type=(jax.ShapeDtypeStruct((B, 2 * D), jnp.int32),
                  jax.ShapeDtypeStruct((B, 2 * D), jnp.int32)),
        mesh=mesh,
        scratch_types=[
            pltpu.VMEM((NCH, CH), jnp.int32),
            pltpu.VMEM((NCH, CH), jnp.int32),
            pltpu.VMEM((BPW, 2 * D), jnp.int32),
            pltpu.SemaphoreType.DMA,
        ],
    )
    return f(uq, mq, ut_pack, mt_pack)


BB = 2048


def _unpack(xi, sel):
    half = jnp.where((sel >> 1) > 0, 1, 0)
    xa = jnp.where(half > 0, xi[:, D:], xi[:, :D])
    lo = lax.bitcast_convert_type(lax.shift_left(xa, 16), jnp.float32)
    hi = lax.bitcast_convert_type(xa & HI, jnp.float32)
    return jnp.where((sel & 1) > 0, hi, lo).astype(jnp.bfloat16)


def _mlp_body(w1u_ref, w1m_ref, b1_ref, w2_ref, b2_ref, w3_ref, b3_ref,
              up_ref, mp_ref, su_ref, sm_ref, out_ref):
    u = _unpack(up_ref[...], su_ref[...])
    m = _unpack(mp_ref[...], sm_ref[...])
    h = jnp.dot(u, w1u_ref[...], preferred_element_type=jnp.float32)
    h = h + jnp.dot(m, w1m_ref[...], preferred_element_type=jnp.float32)
    h = jnp.maximum(h + b1_ref[...], 0.0)
    h = jnp.dot(h.astype(jnp.bfloat16), w2_ref[...],
                preferred_element_type=jnp.float32)
    h = jnp.maximum(h + b2_ref[...], 0.0)
    o = jnp.sum(h * w3_ref[...], axis=1, keepdims=True) + b3_ref[...]
    out_ref[...] = MIN_RATING + (MAX_RATING - MIN_RATING) * jax.nn.sigmoid(o)


def _mlp(up, mp, su, sm, w1u, w1m, b1, w2, b2, w3, b3):
    grid = B // BB
    wspec = lambda shape: pl.BlockSpec(shape, lambda i: (0, 0))
    return pl.pallas_call(
        _mlp_body,
        grid=(grid,),
        in_specs=[
            wspec((D, H1)), wspec((D, H1)), wspec((1, H1)),
            wspec((H1, H2)), wspec((1, H2)), wspec((1, H2)), wspec((1, 1)),
            pl.BlockSpec((BB, 2 * D), lambda i: (i, 0)),
            pl.BlockSpec((BB, 2 * D), lambda i: (i, 0)),
            pl.BlockSpec((BB, 1), lambda i: (i, 0)),
            pl.BlockSpec((BB, 1), lambda i: (i, 0)),
        ],
        out_specs=pl.BlockSpec((BB, 1), lambda i: (i, 0)),
        out_shape=jax.ShapeDtypeStruct((B, 1), jnp.float32),
    )(w1u, w1m, b1, w2, b2, w3, b3, up, mp, su, sm)


def kernel(user_ids, movie_ids, user_table, movie_table, W1, b1, W2, b2, W3, b3):
    uid = user_ids.astype(jnp.int32)
    mid = movie_ids.astype(jnp.int32)
    uq = (((uid >> 12) << 10) | (uid & 1023)).reshape(NW, NCH, CH)
    mq = (((mid >> 12) << 10) | (mid & 1023)).reshape(NW, NCH, CH)
    su = ((uid >> 10) & 3).reshape(B, 1)
    sm = ((mid >> 10) & 3).reshape(B, 1)
    ut_pack = _repack(user_table.T, user_table.shape[0])
    mt_pack = _repack(movie_table.T, movie_table.shape[0])
    up, mp = _sc_gather(uq, mq, ut_pack, mt_pack)
    w1u = W1[:, :D].T.astype(jnp.bfloat16)
    w1m = W1[:, D:].T.astype(jnp.bfloat16)
    out = _mlp(up, mp, su, sm, w1u, w1m, b1.reshape(1, H1),
               W2.T.astype(jnp.bfloat16), b2.reshape(1, H2), W3, b3.reshape(1, 1))
    return out.reshape(B)


# speedup vs baseline: 3.2015x; 1.2581x over previous
import jax
import jax.numpy as jnp
from jax import lax
from jax.experimental import pallas as pl
from jax.experimental.pallas import tpu as pltpu
from jax.experimental.pallas import tpu_sc as plsc

B = 16384
D = 64
H1 = 256
H2 = 128
NC = 2
NS = 16
NW = NC * NS
BPW = B // NW
CH = 128
NCH = BPW // CH

MIN_RATING = 0.5
MAX_RATING = 5.0

RC = 8192
RQ = RC // 4
HI = -65536


def _repack_body(in_ref, out_ref):
    blk = in_ref[...].astype(jnp.bfloat16)
    r = lax.broadcasted_iota(jnp.int32, (D, D), 0)
    c = lax.broadcasted_iota(jnp.int32, (D, D), 1)
    eye = (r == c).astype(jnp.bfloat16)
    t = lax.dot_general(blk, eye, (((0,), (0,)), ((), ())),
                        preferred_element_type=jnp.float32)
    ti = lax.bitcast_convert_type(t, jnp.int32)
    q0 = ti[0 * RQ:1 * RQ]
    q1 = ti[1 * RQ:2 * RQ]
    q2 = ti[2 * RQ:3 * RQ]
    q3 = ti[3 * RQ:4 * RQ]
    p01 = (q1 & HI) | lax.shift_right_logical(q0, 16)
    p23 = (q3 & HI) | lax.shift_right_logical(q2, 16)
    out_ref[...] = jnp.concatenate([p01, p23], axis=1)


def _repack(tab_t, n_rows):
    grid = (n_rows + RC - 1) // RC
    return pl.pallas_call(
        _repack_body,
        grid=(grid,),
        in_specs=[pl.BlockSpec((D, RC), lambda i: (0, i))],
        out_specs=pl.BlockSpec((RQ, 2 * D), lambda i: (i, 0)),
        out_shape=jax.ShapeDtypeStruct((n_rows // 4, 2 * D), jnp.int32),
    )(tab_t)


def _sc_gather_body(uid_hbm, mid_hbm, ut_hbm, mt_hbm, uo_hbm, mo_hbm,
                    uidx_v, midx_v, rows_v, usem):
    wid = lax.axis_index("s") * NC + lax.axis_index("c")
    base = wid * BPW
    pltpu.sync_copy(uid_hbm.at[wid], uidx_v)
    pltpu.sync_copy(mid_hbm.at[wid], midx_v)
    copies = []
    for c in range(NCH):
        copies.append(pltpu.async_copy(
            ut_hbm.at[uidx_v.at[c]], rows_v.at[pl.ds(c * CH, CH)], usem))
    for cp in copies:
        cp.wait()
    pltpu.sync_copy(rows_v, uo_hbm.at[pl.ds(base, BPW)])
    copies = []
    for c in range(NCH):
        copies.append(pltpu.async_copy(
            mt_hbm.at[midx_v.at[c]], rows_v.at[pl.ds(c * CH, CH)], usem))
    for cp in copies:
        cp.wait()
    pltpu.sync_copy(rows_v, mo_hbm.at[pl.ds(base, BPW)])


def _sc_gather(uq, mq, ut_pack, mt_pack):
    mesh = plsc.VectorSubcoreMesh(
        core_axis_name="c", subcore_axis_name="s",
        num_cores=NC, num_subcores=NS)
    f = pl.kernel(
        _sc_gather_body,
        out_type=(jax.ShapeDtypeStruct((B, 2 * D), jnp.int32),
                  jax.ShapeDtypeStruct((B, 2 * D), jnp.int32)),
        mesh=mesh,
        scratch_types=[
            pltpu.VMEM((NCH, CH), jnp.int32),
            pltpu.VMEM((NCH, CH), jnp.int32),
            pltpu.VMEM((BPW, 2 * D), jnp.int32),
            pltpu.SemaphoreType.DMA,
        ],
    )
    return f(uq, mq, ut_pack, mt_pack)


BB = 2048


def _unpack(xi, sel):
    half = jnp.where((sel >> 1) > 0, 1, 0)
    xa = jnp.where(half > 0, xi[:, D:], xi[:, :D])
    lo = lax.bitcast_convert_type(lax.shift_left(xa, 16), jnp.float32)
    hi = lax.bitcast_convert_type(xa & HI, jnp.float32)
    return jnp.where((sel & 1) > 0, hi, lo).astype(jnp.bfloat16)


def _mlp_body(w1u_ref, w1m_ref, b1_ref, w2_ref, b2_ref, w3_ref, b3_ref,
              up_ref, mp_ref, su_ref, sm_ref, out_ref):
    u = _unpack(up_ref[...], su_ref[...])
    m = _unpack(mp_ref[...], sm_ref[...])
    h = jnp.dot(u, w1u_ref[...], preferred_element_type=jnp.float32)
    h = h + jnp.dot(m, w1m_ref[...], preferred_element_type=jnp.float32)
    h = jnp.maximum(h + b1_ref[...], 0.0)
    h = jnp.dot(h.astype(jnp.bfloat16), w2_ref[...],
                preferred_element_type=jnp.float32)
    h = jnp.maximum(h + b2_ref[...], 0.0)
    o = jnp.sum(h * w3_ref[...], axis=1, keepdims=True) + b3_ref[...]
    out_ref[...] = MIN_RATING + (MAX_RATING - MIN_RATING) * jax.nn.sigmoid(o)


def _mlp(up, mp, su, sm, w1u, w1m, b1, w2, b2, w3, b3):
    grid = B // BB
    wspec = lambda shape: pl.BlockSpec(shape, lambda i: (0, 0))
    return pl.pallas_call(
        _mlp_body,
        grid=(grid,),
        in_specs=[
            wspec((D, H1)), wspec((D, H1)), wspec((1, H1)),
            wspec((H1, H2)), wspec((1, H2)), wspec((1, H2)), wspec((1, 1)),
            pl.BlockSpec((BB, 2 * D), lambda i: (i, 0)),
            pl.BlockSpec((BB, 2 * D), lambda i: (i, 0)),
            pl.BlockSpec((BB, 1), lambda i: (i, 0)),
            pl.BlockSpec((BB, 1), lambda i: (i, 0)),
        ],
        out_specs=pl.BlockSpec((BB, 1), lambda i: (i, 0)),
        out_shape=jax.ShapeDtypeStruct((B, 1), jnp.float32),
    )(w1u, w1m, b1, w2, b2, w3, b3, up, mp, su, sm)


def kernel(user_ids, movie_ids, user_table, movie_table, W1, b1, W2, b2, W3, b3):
    uid = user_ids.astype(jnp.int32)
    mid = movie_ids.astype(jnp.int32)
    uq = (((uid >> 13) << 11) | (uid & 2047)).reshape(NW, NCH, CH)
    mq = (((mid >> 13) << 11) | (mid & 2047)).reshape(NW, NCH, CH)
    su = ((uid >> 11) & 3).reshape(B, 1)
    sm = ((mid >> 11) & 3).reshape(B, 1)
    ut_pack = _repack(user_table.T, user_table.shape[0])
    mt_pack = _repack(movie_table.T, movie_table.shape[0])
    up, mp = _sc_gather(uq, mq, ut_pack, mt_pack)
    w1u = W1[:, :D].T.astype(jnp.bfloat16)
    w1m = W1[:, D:].T.astype(jnp.bfloat16)
    out = _mlp(up, mp, su, sm, w1u, w1m, b1.reshape(1, H1),
               W2.T.astype(jnp.bfloat16), b2.reshape(1, H2), W3, b3.reshape(1, 1))
    return out.reshape(B)


# speedup vs baseline: 3.7409x; 1.1685x over previous
import jax
import jax.numpy as jnp
from jax import lax
from jax.experimental import pallas as pl
from jax.experimental.pallas import tpu as pltpu
from jax.experimental.pallas import tpu_sc as plsc

B = 16384
D = 64
H1 = 256
H2 = 128
NC = 2
NS = 16
NW = NC * NS
BPW = B // NW
CH = 128
NCH = BPW // CH

MIN_RATING = 0.5
MAX_RATING = 5.0

RC = 16384
RQ = RC // 4
HI = -65536


def _repack_body(in_ref, out_ref):
    blk = in_ref[...].astype(jnp.bfloat16)
    r = lax.broadcasted_iota(jnp.int32, (D, D), 0)
    c = lax.broadcasted_iota(jnp.int32, (D, D), 1)
    eye = (r == c).astype(jnp.bfloat16)
    t = lax.dot_general(blk, eye, (((0,), (0,)), ((), ())),
                        preferred_element_type=jnp.float32)
    ti = lax.bitcast_convert_type(t, jnp.int32)
    q0 = ti[0 * RQ:1 * RQ]
    q1 = ti[1 * RQ:2 * RQ]
    q2 = ti[2 * RQ:3 * RQ]
    q3 = ti[3 * RQ:4 * RQ]
    p01 = (q1 & HI) | lax.shift_right_logical(q0, 16)
    p23 = (q3 & HI) | lax.shift_right_logical(q2, 16)
    out_ref[...] = jnp.concatenate([p01, p23], axis=1)


def _repack(tab_t, n_rows):
    grid = (n_rows + RC - 1) // RC
    return pl.pallas_call(
        _repack_body,
        grid=(grid,),
        in_specs=[pl.BlockSpec((D, RC), lambda i: (0, i))],
        out_specs=pl.BlockSpec((RQ, 2 * D), lambda i: (i, 0)),
        out_shape=jax.ShapeDtypeStruct((n_rows // 4, 2 * D), jnp.int32),
    )(tab_t)


def _sc_gather_body(uid_hbm, mid_hbm, ut_hbm, mt_hbm, uo_hbm, mo_hbm,
                    uidx_v, midx_v, rows_v, usem):
    wid = lax.axis_index("s") * NC + lax.axis_index("c")
    base = wid * BPW
    pltpu.sync_copy(uid_hbm.at[wid], uidx_v)
    pltpu.sync_copy(mid_hbm.at[wid], midx_v)
    copies = []
    for c in range(NCH):
        copies.append(pltpu.async_copy(
            ut_hbm.at[uidx_v.at[c]], rows_v.at[pl.ds(c * CH, CH)], usem))
    for cp in copies:
        cp.wait()
    pltpu.sync_copy(rows_v, uo_hbm.at[pl.ds(base, BPW)])
    copies = []
    for c in range(NCH):
        copies.append(pltpu.async_copy(
            mt_hbm.at[midx_v.at[c]], rows_v.at[pl.ds(c * CH, CH)], usem))
    for cp in copies:
        cp.wait()
    pltpu.sync_copy(rows_v, mo_hbm.at[pl.ds(base, BPW)])


def _sc_gather(uq, mq, ut_pack, mt_pack):
    mesh = plsc.VectorSubcoreMesh(
        core_axis_name="c", subcore_axis_name="s",
        num_cores=NC, num_subcores=NS)
    f = pl.kernel(
        _sc_gather_body,
        out_type=(jax.ShapeDtypeStruct((B, 2 * D), jnp.int32),
                  jax.ShapeDtypeStruct((B, 2 * D), jnp.int32)),
        mesh=mesh,
        scratch_types=[
            pltpu.VMEM((NCH, CH), jnp.int32),
            pltpu.VMEM((NCH, CH), jnp.int32),
            pltpu.VMEM((BPW, 2 * D), jnp.int32),
            pltpu.SemaphoreType.DMA,
        ],
    )
    return f(uq, mq, ut_pack, mt_pack)


BB = 2048


def _unpack(xi, sel):
    half = jnp.where((sel >> 1) > 0, 1, 0)
    xa = jnp.where(half > 0, xi[:, D:], xi[:, :D])
    lo = lax.bitcast_convert_type(lax.shift_left(xa, 16), jnp.float32)
    hi = lax.bitcast_convert_type(xa & HI, jnp.float32)
    return jnp.where((sel & 1) > 0, hi, lo).astype(jnp.bfloat16)


def _mlp_body(w1u_ref, w1m_ref, b1_ref, w2_ref, b2_ref, w3_ref, b3_ref,
              up_ref, mp_ref, su_ref, sm_ref, out_ref):
    u = _unpack(up_ref[...], su_ref[...])
    m = _unpack(mp_ref[...], sm_ref[...])
    h = jnp.dot(u, w1u_ref[...], preferred_element_type=jnp.float32)
    h = h + jnp.dot(m, w1m_ref[...], preferred_element_type=jnp.float32)
    h = jnp.maximum(h + b1_ref[...], 0.0)
    h = jnp.dot(h.astype(jnp.bfloat16), w2_ref[...],
                preferred_element_type=jnp.float32)
    h = jnp.maximum(h + b2_ref[...], 0.0)
    o = jnp.sum(h * w3_ref[...], axis=1, keepdims=True) + b3_ref[...]
    out_ref[...] = MIN_RATING + (MAX_RATING - MIN_RATING) * jax.nn.sigmoid(o)


def _mlp(up, mp, su, sm, w1u, w1m, b1, w2, b2, w3, b3):
    grid = B // BB
    wspec = lambda shape: pl.BlockSpec(shape, lambda i: (0, 0))
    return pl.pallas_call(
        _mlp_body,
        grid=(grid,),
        in_specs=[
            wspec((D, H1)), wspec((D, H1)), wspec((1, H1)),
            wspec((H1, H2)), wspec((1, H2)), wspec((1, H2)), wspec((1, 1)),
            pl.BlockSpec((BB, 2 * D), lambda i: (i, 0)),
            pl.BlockSpec((BB, 2 * D), lambda i: (i, 0)),
            pl.BlockSpec((BB, 1), lambda i: (i, 0)),
            pl.BlockSpec((BB, 1), lambda i: (i, 0)),
        ],
        out_specs=pl.BlockSpec((BB, 1), lambda i: (i, 0)),
        out_shape=jax.ShapeDtypeStruct((B, 1), jnp.float32),
    )(w1u, w1m, b1, w2, b2, w3, b3, up, mp, su, sm)


def kernel(user_ids, movie_ids, user_table, movie_table, W1, b1, W2, b2, W3, b3):
    uid = user_ids.astype(jnp.int32)
    mid = movie_ids.astype(jnp.int32)
    uq = (((uid >> 14) << 12) | (uid & 4095)).reshape(NW, NCH, CH)
    mq = (((mid >> 14) << 12) | (mid & 4095)).reshape(NW, NCH, CH)
    su = ((uid >> 12) & 3).reshape(B, 1)
    sm = ((mid >> 12) & 3).reshape(B, 1)
    ut_pack = _repack(user_table.T, user_table.shape[0])
    mt_pack = _repack(movie_table.T, movie_table.shape[0])
    up, mp = _sc_gather(uq, mq, ut_pack, mt_pack)
    w1u = W1[:, :D].T.astype(jnp.bfloat16)
    w1m = W1[:, D:].T.astype(jnp.bfloat16)
    out = _mlp(up, mp, su, sm, w1u, w1m, b1.reshape(1, H1),
               W2.T.astype(jnp.bfloat16), b2.reshape(1, H2), W3, b3.reshape(1, 1))
    return out.reshape(B)


# speedup vs baseline: 3.7860x; 1.0121x over previous
import jax
import jax.numpy as jnp
from jax import lax
from jax.experimental import pallas as pl
from jax.experimental.pallas import tpu as pltpu
from jax.experimental.pallas import tpu_sc as plsc

B = 16384
D = 64
H1 = 256
H2 = 128
NC = 2
NS = 16
NW = NC * NS
BPW = B // NW
CH = 128
NCH = BPW // CH

MIN_RATING = 0.5
MAX_RATING = 5.0

RC = 16384
RQ = RC // 4
HI = -65536


def _repack_body(in_ref, out_ref):
    blk = in_ref[...].astype(jnp.bfloat16)
    r = lax.broadcasted_iota(jnp.int32, (D, D), 0)
    c = lax.broadcasted_iota(jnp.int32, (D, D), 1)
    eye = (r == c).astype(jnp.bfloat16)
    t = lax.dot_general(blk, eye, (((0,), (0,)), ((), ())),
                        preferred_element_type=jnp.float32)
    ti = lax.bitcast_convert_type(t, jnp.int32)
    q0 = ti[0 * RQ:1 * RQ]
    q1 = ti[1 * RQ:2 * RQ]
    q2 = ti[2 * RQ:3 * RQ]
    q3 = ti[3 * RQ:4 * RQ]
    p01 = (q1 & HI) | lax.shift_right_logical(q0, 16)
    p23 = (q3 & HI) | lax.shift_right_logical(q2, 16)
    out_ref[...] = jnp.concatenate([p01, p23], axis=1)


def _repack(tab_t, n_rows):
    grid = (n_rows + RC - 1) // RC
    return pl.pallas_call(
        _repack_body,
        grid=(grid,),
        in_specs=[pl.BlockSpec((D, RC), lambda i: (0, i))],
        out_specs=pl.BlockSpec((RQ, 2 * D), lambda i: (i, 0)),
        out_shape=jax.ShapeDtypeStruct((n_rows // 4, 2 * D), jnp.int32),
    )(tab_t)


def _sc_gather_body(idx_hbm, tab_hbm, out_hbm, idx_v, rows_v, sem):
    wid = lax.axis_index("s") * NC + lax.axis_index("c")
    base = wid * BPW
    pltpu.sync_copy(idx_hbm.at[wid], idx_v)
    copies = []
    for c in range(NCH):
        copies.append(pltpu.async_copy(
            tab_hbm.at[idx_v.at[c]], rows_v.at[pl.ds(c * CH, CH)], sem))
    for cp in copies:
        cp.wait()
    pltpu.sync_copy(rows_v, out_hbm.at[pl.ds(base, BPW)])


def _sc_gather(idx, tab_pack):
    mesh = plsc.VectorSubcoreMesh(
        core_axis_name="c", subcore_axis_name="s",
        num_cores=NC, num_subcores=NS)
    f = pl.kernel(
        _sc_gather_body,
        out_type=jax.ShapeDtypeStruct((B, 2 * D), jnp.int32),
        mesh=mesh,
        scratch_types=[
            pltpu.VMEM((NCH, CH), jnp.int32),
            pltpu.VMEM((BPW, 2 * D), jnp.int32),
            pltpu.SemaphoreType.DMA,
        ],
    )
    return f(idx, tab_pack)


BB = 2048


def _unpack(xi, sel):
    half = jnp.where((sel >> 1) > 0, 1, 0)
    xa = jnp.where(half > 0, xi[:, D:], xi[:, :D])
    lo = lax.bitcast_convert_type(lax.shift_left(xa, 16), jnp.float32)
    hi = lax.bitcast_convert_type(xa & HI, jnp.float32)
    return jnp.where((sel & 1) > 0, hi, lo).astype(jnp.bfloat16)


def _mlp_body(w1u_ref, w1m_ref, b1_ref, w2_ref, b2_ref, w3_ref, b3_ref,
              up_ref, mp_ref, su_ref, sm_ref, out_ref):
    u = _unpack(up_ref[...], su_ref[...])
    m = _unpack(mp_ref[...], sm_ref[...])
    h = jnp.dot(u, w1u_ref[...], preferred_element_type=jnp.float32)
    h = h + jnp.dot(m, w1m_ref[...], preferred_element_type=jnp.float32)
    h = jnp.maximum(h + b1_ref[...], 0.0)
    h = jnp.dot(h.astype(jnp.bfloat16), w2_ref[...],
                preferred_element_type=jnp.float32)
    h = jnp.maximum(h + b2_ref[...], 0.0)
    o = jnp.sum(h * w3_ref[...], axis=1, keepdims=True) + b3_ref[...]
    out_ref[...] = MIN_RATING + (MAX_RATING - MIN_RATING) * jax.nn.sigmoid(o)


def _mlp(up, mp, su, sm, w1u, w1m, b1, w2, b2, w3, b3):
    grid = B // BB
    wspec = lambda shape: pl.BlockSpec(shape, lambda i: (0, 0))
    return pl.pallas_call(
        _mlp_body,
        grid=(grid,),
        in_specs=[
            wspec((D, H1)), wspec((D, H1)), wspec((1, H1)),
            wspec((H1, H2)), wspec((1, H2)), wspec((1, H2)), wspec((1, 1)),
            pl.BlockSpec((BB, 2 * D), lambda i: (i, 0)),
            pl.BlockSpec((BB, 2 * D), lambda i: (i, 0)),
            pl.BlockSpec((BB, 1), lambda i: (i, 0)),
            pl.BlockSpec((BB, 1), lambda i: (i, 0)),
        ],
        out_specs=pl.BlockSpec((BB, 1), lambda i: (i, 0)),
        out_shape=jax.ShapeDtypeStruct((B, 1), jnp.float32),
    )(w1u, w1m, b1, w2, b2, w3, b3, up, mp, su, sm)


def kernel(user_ids, movie_ids, user_table, movie_table, W1, b1, W2, b2, W3, b3):
    uid = user_ids.astype(jnp.int32)
    mid = movie_ids.astype(jnp.int32)
    uq = (((uid >> 14) << 12) | (uid & 4095)).reshape(NW, NCH, CH)
    mq = (((mid >> 14) << 12) | (mid & 4095)).reshape(NW, NCH, CH)
    su = ((uid >> 12) & 3).reshape(B, 1)
    sm = ((mid >> 12) & 3).reshape(B, 1)
    mt_pack = _repack(movie_table.T, movie_table.shape[0])
    mp = _sc_gather(mq, mt_pack)
    ut_pack = _repack(user_table.T, user_table.shape[0])
    up = _sc_gather(uq, ut_pack)
    w1u = W1[:, :D].T.astype(jnp.bfloat16)
    w1m = W1[:, D:].T.astype(jnp.bfloat16)
    out = _mlp(up, mp, su, sm, w1u, w1m, b1.reshape(1, H1),
               W2.T.astype(jnp.bfloat16), b2.reshape(1, H2), W3, b3.reshape(1, 1))
    return out.reshape(B)


# speedup vs baseline: 4.1113x; 1.0859x over previous
import jax
import jax.numpy as jnp
from jax import lax
from jax.experimental import pallas as pl
from jax.experimental.pallas import tpu as pltpu
from jax.experimental.pallas import tpu_sc as plsc

B = 16384
D = 64
H1 = 256
H2 = 128
NC = 2
NS = 16
NW = NC * NS
BPW = B // NW
CH = 128
NCH = BPW // CH

MIN_RATING = 0.5
MAX_RATING = 5.0

RC = 32768
RQ = RC // 4
HI = -65536


def _repack_body(in_ref, out_ref):
    blk = in_ref[...].astype(jnp.bfloat16)
    r = lax.broadcasted_iota(jnp.int32, (D, D), 0)
    c = lax.broadcasted_iota(jnp.int32, (D, D), 1)
    eye = (r == c).astype(jnp.bfloat16)
    t = lax.dot_general(blk, eye, (((0,), (0,)), ((), ())),
                        preferred_element_type=jnp.float32)
    ti = lax.bitcast_convert_type(t, jnp.int32)
    q0 = ti[0 * RQ:1 * RQ]
    q1 = ti[1 * RQ:2 * RQ]
    q2 = ti[2 * RQ:3 * RQ]
    q3 = ti[3 * RQ:4 * RQ]
    p01 = (q1 & HI) | lax.shift_right_logical(q0, 16)
    p23 = (q3 & HI) | lax.shift_right_logical(q2, 16)
    out_ref[...] = jnp.concatenate([p01, p23], axis=1)


def _repack(tab_t, n_rows):
    grid = (n_rows + RC - 1) // RC
    return pl.pallas_call(
        _repack_body,
        grid=(grid,),
        in_specs=[pl.BlockSpec((D, RC), lambda i: (0, i))],
        out_specs=pl.BlockSpec((RQ, 2 * D), lambda i: (i, 0)),
        out_shape=jax.ShapeDtypeStruct((n_rows // 4, 2 * D), jnp.int32),
    )(tab_t)


def _sc_gather_body(idx_hbm, tab_hbm, out_hbm, idx_v, rows_v, sem):
    wid = lax.axis_index("s") * NC + lax.axis_index("c")
    base = wid * BPW
    pltpu.sync_copy(idx_hbm.at[wid], idx_v)
    copies = []
    for c in range(NCH):
        copies.append(pltpu.async_copy(
            tab_hbm.at[idx_v.at[c]], rows_v.at[pl.ds(c * CH, CH)], sem))
    for cp in copies:
        cp.wait()
    pltpu.sync_copy(rows_v, out_hbm.at[pl.ds(base, BPW)])


def _sc_gather(idx, tab_pack):
    mesh = plsc.VectorSubcoreMesh(
        core_axis_name="c", subcore_axis_name="s",
        num_cores=NC, num_subcores=NS)
    f = pl.kernel(
        _sc_gather_body,
        out_type=jax.ShapeDtypeStruct((B, 2 * D), jnp.int32),
        mesh=mesh,
        scratch_types=[
            pltpu.VMEM((NCH, CH), jnp.int32),
            pltpu.VMEM((BPW, 2 * D), jnp.int32),
            pltpu.SemaphoreType.DMA,
        ],
    )
    return f(idx, tab_pack)


BB = 2048


def _unpack(xi, sel):
    half = jnp.where((sel >> 1) > 0, 1, 0)
    xa = jnp.where(half > 0, xi[:, D:], xi[:, :D])
    lo = lax.bitcast_convert_type(lax.shift_left(xa, 16), jnp.float32)
    hi = lax.bitcast_convert_type(xa & HI, jnp.float32)
    return jnp.where((sel & 1) > 0, hi, lo).astype(jnp.bfloat16)


def _mlp_body(w1u_ref, w1m_ref, b1_ref, w2_ref, b2_ref, w3_ref, b3_ref,
              up_ref, mp_ref, su_ref, sm_ref, out_ref):
    u = _unpack(up_ref[...], su_ref[...])
    m = _unpack(mp_ref[...], sm_ref[...])
    h = jnp.dot(u, w1u_ref[...], preferred_element_type=jnp.float32)
    h = h + jnp.dot(m, w1m_ref[...], preferred_element_type=jnp.float32)
    h = jnp.maximum(h + b1_ref[...], 0.0)
    h = jnp.dot(h.astype(jnp.bfloat16), w2_ref[...],
                preferred_element_type=jnp.float32)
    h = jnp.maximum(h + b2_ref[...], 0.0)
    o = jnp.sum(h * w3_ref[...], axis=1, keepdims=True) + b3_ref[...]
    out_ref[...] = MIN_RATING + (MAX_RATING - MIN_RATING) * jax.nn.sigmoid(o)


def _mlp(up, mp, su, sm, w1u, w1m, b1, w2, b2, w3, b3):
    grid = B // BB
    wspec = lambda shape: pl.BlockSpec(shape, lambda i: (0, 0))
    return pl.pallas_call(
        _mlp_body,
        grid=(grid,),
        in_specs=[
            wspec((D, H1)), wspec((D, H1)), wspec((1, H1)),
            wspec((H1, H2)), wspec((1, H2)), wspec((1, H2)), wspec((1, 1)),
            pl.BlockSpec((BB, 2 * D), lambda i: (i, 0)),
            pl.BlockSpec((BB, 2 * D), lambda i: (i, 0)),
            pl.BlockSpec((BB, 1), lambda i: (i, 0)),
            pl.BlockSpec((BB, 1), lambda i: (i, 0)),
        ],
        out_specs=pl.BlockSpec((BB, 1), lambda i: (i, 0)),
        out_shape=jax.ShapeDtypeStruct((B, 1), jnp.float32),
    )(w1u, w1m, b1, w2, b2, w3, b3, up, mp, su, sm)


def kernel(user_ids, movie_ids, user_table, movie_table, W1, b1, W2, b2, W3, b3):
    uid = user_ids.astype(jnp.int32)
    mid = movie_ids.astype(jnp.int32)
    uq = (((uid >> 14) << 12) | (uid & 4095)).reshape(NW, NCH, CH)
    mq = (((mid >> 14) << 12) | (mid & 4095)).reshape(NW, NCH, CH)
    su = ((uid >> 12) & 3).reshape(B, 1)
    sm = ((mid >> 12) & 3).reshape(B, 1)
    mt_pack = _repack(movie_table.T, movie_table.shape[0])
    mp = _sc_gather(mq, mt_pack)
    ut_pack = _repack(user_table.T, user_table.shape[0])
    up = _sc_gather(uq, ut_pack)
    w1u = W1[:, :D].T.astype(jnp.bfloat16)
    w1m = W1[:, D:].T.astype(jnp.bfloat16)
    out = _mlp(up, mp, su, sm, w1u, w1m, b1.reshape(1, H1),
               W2.T.astype(jnp.bfloat16), b2.reshape(1, H2), W3, b3.reshape(1, 1))
    return out.reshape(B)


# speedup vs baseline: 4.1304x; 1.0047x over previous
import jax
import jax.numpy as jnp
from jax import lax
from jax.experimental import pallas as pl
from jax.experimental.pallas import tpu as pltpu
from jax.experimental.pallas import tpu_sc as plsc

B = 16384
D = 64
H1 = 256
H2 = 128
NC = 2
NS = 16
NW = NC * NS
BPW = B // NW
CH = 128
NCH = BPW // CH

MIN_RATING = 0.5
MAX_RATING = 5.0

RC = 49152
RQ = RC // 4
HI = -65536


def _repack_body(in_ref, out_ref):
    blk = in_ref[...].astype(jnp.bfloat16)
    r = lax.broadcasted_iota(jnp.int32, (D, D), 0)
    c = lax.broadcasted_iota(jnp.int32, (D, D), 1)
    eye = (r == c).astype(jnp.bfloat16)
    t = lax.dot_general(blk, eye, (((0,), (0,)), ((), ())),
                        preferred_element_type=jnp.float32)
    ti = lax.bitcast_convert_type(t, jnp.int32)
    q0 = ti[0 * RQ:1 * RQ]
    q1 = ti[1 * RQ:2 * RQ]
    q2 = ti[2 * RQ:3 * RQ]
    q3 = ti[3 * RQ:4 * RQ]
    p01 = (q1 & HI) | lax.shift_right_logical(q0, 16)
    p23 = (q3 & HI) | lax.shift_right_logical(q2, 16)
    out_ref[...] = jnp.concatenate([p01, p23], axis=1)


def _repack(tab_t, n_rows):
    grid = (n_rows + RC - 1) // RC
    return pl.pallas_call(
        _repack_body,
        grid=(grid,),
        in_specs=[pl.BlockSpec((D, RC), lambda i: (0, i))],
        out_specs=pl.BlockSpec((RQ, 2 * D), lambda i: (i, 0)),
        out_shape=jax.ShapeDtypeStruct((n_rows // 4, 2 * D), jnp.int32),
    )(tab_t)


def _sc_gather_body(idx_hbm, tab_hbm, out_hbm, idx_v, rows_v, sem):
    wid = lax.axis_index("s") * NC + lax.axis_index("c")
    base = wid * BPW
    pltpu.sync_copy(idx_hbm.at[wid], idx_v)
    copies = []
    for c in range(NCH):
        copies.append(pltpu.async_copy(
            tab_hbm.at[idx_v.at[c]], rows_v.at[pl.ds(c * CH, CH)], sem))
    for cp in copies:
        cp.wait()
    pltpu.sync_copy(rows_v, out_hbm.at[pl.ds(base, BPW)])


def _sc_gather(idx, tab_pack):
    mesh = plsc.VectorSubcoreMesh(
        core_axis_name="c", subcore_axis_name="s",
        num_cores=NC, num_subcores=NS)
    f = pl.kernel(
        _sc_gather_body,
        out_type=jax.ShapeDtypeStruct((B, 2 * D), jnp.int32),
        mesh=mesh,
        scratch_types=[
            pltpu.VMEM((NCH, CH), jnp.int32),
            pltpu.VMEM((BPW, 2 * D), jnp.int32),
            pltpu.SemaphoreType.DMA,
        ],
    )
    return f(idx, tab_pack)


BB = 2048


def _unpack(xi, rid):
    sel = (rid >> 12) & 3
    half = jnp.where((sel >> 1) > 0, 1, 0)
    xa = jnp.where(half > 0, xi[:, D:], xi[:, :D])
    lo = lax.bitcast_convert_type(lax.shift_left(xa, 16), jnp.float32)
    hi = lax.bitcast_convert_type(xa & HI, jnp.float32)
    return jnp.where((sel & 1) > 0, hi, lo).astype(jnp.bfloat16)


def _mlp_body(w1u_ref, w1m_ref, b1_ref, w2_ref, b2_ref, w3_ref, b3_ref,
              up_ref, mp_ref, su_ref, sm_ref, out_ref):
    u = _unpack(up_ref[...], su_ref[...])
    m = _unpack(mp_ref[...], sm_ref[...])
    h = jnp.dot(u, w1u_ref[...], preferred_element_type=jnp.float32)
    h = h + jnp.dot(m, w1m_ref[...], preferred_element_type=jnp.float32)
    h = jnp.maximum(h + b1_ref[...], 0.0)
    h = jnp.dot(h.astype(jnp.bfloat16), w2_ref[...],
                preferred_element_type=jnp.float32)
    h = jnp.maximum(h + b2_ref[...], 0.0)
    o = jnp.sum(h * w3_ref[...], axis=1, keepdims=True) + b3_ref[...]
    out_ref[...] = MIN_RATING + (MAX_RATING - MIN_RATING) * jax.nn.sigmoid(o)


def _mlp(up, mp, su, sm, w1u, w1m, b1, w2, b2, w3, b3):
    grid = B // BB
    wspec = lambda shape: pl.BlockSpec(shape, lambda i: (0, 0))
    return pl.pallas_call(
        _mlp_body,
        grid=(grid,),
        in_specs=[
            wspec((D, H1)), wspec((D, H1)), wspec((1, H1)),
            wspec((H1, H2)), wspec((1, H2)), wspec((1, H2)), wspec((1, 1)),
            pl.BlockSpec((BB, 2 * D), lambda i: (i, 0)),
            pl.BlockSpec((BB, 2 * D), lambda i: (i, 0)),
            pl.BlockSpec((BB, 1), lambda i: (i, 0)),
            pl.BlockSpec((BB, 1), lambda i: (i, 0)),
        ],
        out_specs=pl.BlockSpec((BB, 1), lambda i: (i, 0)),
        out_shape=jax.ShapeDtypeStruct((B, 1), jnp.float32),
    )(w1u, w1m, b1, w2, b2, w3, b3, up, mp, su, sm)


def kernel(user_ids, movie_ids, user_table, movie_table, W1, b1, W2, b2, W3, b3):
    uid = user_ids.astype(jnp.int32)
    mid = movie_ids.astype(jnp.int32)
    uq = (((uid >> 14) << 12) | (uid & 4095)).reshape(NW, NCH, CH)
    mq = (((mid >> 14) << 12) | (mid & 4095)).reshape(NW, NCH, CH)
    su = uid.reshape(B, 1)
    sm = mid.reshape(B, 1)
    mt_pack = _repack(movie_table.T, movie_table.shape[0])
    mp = _sc_gather(mq, mt_pack)
    ut_pack = _repack(user_table.T, user_table.shape[0])
    up = _sc_gather(uq, ut_pack)
    w1u = W1[:, :D].T.astype(jnp.bfloat16)
    w1m = W1[:, D:].T.astype(jnp.bfloat16)
    out = _mlp(up, mp, su, sm, w1u, w1m, b1.reshape(1, H1),
               W2.T.astype(jnp.bfloat16), b2.reshape(1, H2), W3, b3.reshape(1, 1))
    return out.reshape(B)
